# baseline (speedup 1.0000x reference)
import functools

import jax
import jax.numpy as jnp
from jax import lax
from jax.experimental import pallas as pl
from jax.experimental.pallas import tpu as pltpu
from jax.experimental.pallas import tpu_sc as plsc

N = 2048
G = 2048
K = 16
G_TILE = 512
NW = 32
WPB = 4
PAIRS_PER_W = (G * K) // WPB
WORDS_PER_W = PAIRS_PER_W * 4


def _knn_idx_body(pcd_ref, pfull_ref, xt_ref, out_ref):
    ct = pcd_ref[0]
    pf = pfull_ref[0]
    xt = xt_ref[0]
    sq = jnp.sum(xt * xt, axis=0, keepdims=True)
    sqg = jnp.sum(ct * ct, axis=1, keepdims=True)
    dots = jax.lax.dot_general(ct, pf, (((1,), (1,)), ((), ())),
                               preferred_element_type=jnp.float32)
    d2 = (sqg + sq) - 2.0 * dots
    ii = lax.broadcasted_iota(jnp.int32, (G_TILE, N), 1)
    idxs = []
    for _ in range(K):
        m = jnp.min(d2, axis=1, keepdims=True)
        cand = jnp.where(d2 == m, ii, N)
        im = jnp.min(cand, axis=1, keepdims=True)
        onehot = ii == im
        idxs.append(im)
        d2 = jnp.where(onehot, jnp.inf, d2)
    out_ref[0] = jnp.concatenate(idxs, axis=1)


def _sc_gather_body(tab_hbm, idx_hbm, out_hbm, tab_v, idx_v, out_v):
    wid = lax.axis_index("s") * 2 + lax.axis_index("c")
    b = wid // WPB
    part = wid % WPB
    pltpu.sync_copy(tab_hbm.at[b], tab_v)
    pltpu.sync_copy(idx_hbm.at[wid], idx_v)
    lanes = lax.broadcasted_iota(jnp.int32, (16,), 0)
    cols = lanes & 3
    quad = lanes >> 2
    gbase = part * PAIRS_PER_W

    def body(i, carry):
        g = (gbase + i * 4) >> 4
        gvec = jnp.broadcast_to(g, (16,))
        rows = plsc.load_gather(idx_v, [jnp.broadcast_to(i * 4, (16,)) + quad])
        val = plsc.load_gather(tab_v, [rows, cols])
        ctr = plsc.load_gather(tab_v, [gvec, cols])
        out_v[pl.ds(i * 16, 16)] = val - ctr
        return carry

    lax.fori_loop(0, PAIRS_PER_W // 4, body, 0)
    pltpu.sync_copy(out_v, out_hbm.at[wid])


@jax.jit
def kernel(pcd):
    B = pcd.shape[0]
    BC = B // 2
    xt = jnp.transpose(pcd, (0, 2, 1))
    tab = jnp.concatenate(
        [pcd, jnp.zeros((B, N, 1), jnp.float32)], axis=2)

    def tc_topk(pc, xtc):
        return pl.pallas_call(
            _knn_idx_body,
            grid=(BC, G // G_TILE),
            in_specs=[
                pl.BlockSpec((1, G_TILE, 3), lambda b, j: (b, j, 0)),
                pl.BlockSpec((1, N, 3), lambda b, j: (b, 0, 0)),
                pl.BlockSpec((1, 3, N), lambda b, j: (b, 0, 0)),
            ],
            out_specs=pl.BlockSpec((1, G_TILE, K), lambda b, j: (b, j, 0)),
            out_shape=jax.ShapeDtypeStruct((BC, G, K), jnp.int32),
        )(pc, pc, xtc)

    mesh = plsc.VectorSubcoreMesh(core_axis_name="c", subcore_axis_name="s")
    sc_gather = functools.partial(
        pl.kernel, mesh=mesh,
        compiler_params=pltpu.CompilerParams(
            needs_layout_passes=False, use_tc_tiling_on_sc=False),
        out_type=jax.ShapeDtypeStruct((NW, WORDS_PER_W), jnp.float32),
        scratch_types=[
            pltpu.VMEM((N, 4), jnp.float32),
            pltpu.VMEM((PAIRS_PER_W,), jnp.int32),
            pltpu.VMEM((WORDS_PER_W,), jnp.float32),
        ],
    )(_sc_gather_body)

    outs = []
    for c in range(2):
        sl = slice(c * BC, (c + 1) * BC)
        idx_c = tc_topk(pcd[sl], xt[sl])
        out_c = sc_gather(tab[sl], idx_c.reshape(NW, PAIRS_PER_W))
        outs.append(out_c.reshape(BC, G, K, 4))

    neighborhood = jnp.concatenate(outs, axis=0)[..., :3]
    return (neighborhood, pcd)


# speedup vs baseline: 1.0449x; 1.0449x over previous
import functools

import jax
import jax.numpy as jnp
from jax import lax
from jax.experimental import pallas as pl
from jax.experimental.pallas import tpu as pltpu
from jax.experimental.pallas import tpu_sc as plsc

N = 2048
G = 2048
K = 16
G_TILE = 512
NW = 32
PAIRS_PER_W = (G * K) // 2
WORDS_PER_W = PAIRS_PER_W * 4


def _knn_idx_body(pcd_ref, pfull_ref, xt_ref, out_ref):
    ct = pcd_ref[0]
    pf = pfull_ref[0]
    xt = xt_ref[0]
    sq = jnp.sum(xt * xt, axis=0, keepdims=True)
    sqg = jnp.sum(ct * ct, axis=1, keepdims=True)
    dots = jax.lax.dot_general(ct, pf, (((1,), (1,)), ((), ())),
                               preferred_element_type=jnp.float32)
    d2 = (sqg + sq) - 2.0 * dots
    ii = lax.broadcasted_iota(jnp.int32, (G_TILE, N), 1)
    idxs = []
    for _ in range(K):
        m = jnp.min(d2, axis=1, keepdims=True)
        cand = jnp.where(d2 == m, ii, N)
        im = jnp.min(cand, axis=1, keepdims=True)
        onehot = ii == im
        idxs.append(im)
        d2 = jnp.where(onehot, jnp.inf, d2)
    out_ref[0] = jnp.concatenate(idxs, axis=1)


def _sc_gather_body(tab_hbm, idx_hbm, out_hbm, tab_v, idx_v, out_v):
    wid = lax.axis_index("s") * 2 + lax.axis_index("c")
    b = wid >> 1
    half = wid & 1
    pltpu.sync_copy(tab_hbm.at[b], tab_v)
    pltpu.sync_copy(idx_hbm.at[wid], idx_v)
    lanes = lax.broadcasted_iota(jnp.int32, (16,), 0)
    cols = lanes & 3
    quad = lanes >> 2
    gbase = half * PAIRS_PER_W

    def body(i, carry):
        g = (gbase + i * 4) >> 4
        gvec = jnp.broadcast_to(g, (16,))
        rows = plsc.load_gather(idx_v, [jnp.broadcast_to(i * 4, (16,)) + quad])
        val = plsc.load_gather(tab_v, [rows, cols])
        ctr = plsc.load_gather(tab_v, [gvec, cols])
        out_v[pl.ds(i * 16, 16)] = val - ctr
        return carry

    lax.fori_loop(0, PAIRS_PER_W // 4, body, 0)
    pltpu.sync_copy(out_v, out_hbm.at[wid])


@jax.jit
def kernel(pcd):
    B = pcd.shape[0]
    xt = jnp.transpose(pcd, (0, 2, 1))
    idx = pl.pallas_call(
        _knn_idx_body,
        grid=(B, G // G_TILE),
        in_specs=[
            pl.BlockSpec((1, G_TILE, 3), lambda b, j: (b, j, 0)),
            pl.BlockSpec((1, N, 3), lambda b, j: (b, 0, 0)),
            pl.BlockSpec((1, 3, N), lambda b, j: (b, 0, 0)),
        ],
        out_specs=pl.BlockSpec((1, G_TILE, K), lambda b, j: (b, j, 0)),
        out_shape=jax.ShapeDtypeStruct((B, G, K), jnp.int32),
    )(pcd, pcd, xt)

    tab = jnp.concatenate(
        [pcd, jnp.zeros((B, N, 1), jnp.float32)], axis=2)
    idx2 = idx.reshape(NW, PAIRS_PER_W)

    mesh = plsc.VectorSubcoreMesh(core_axis_name="c", subcore_axis_name="s")
    sc_gather = functools.partial(
        pl.kernel, mesh=mesh,
        compiler_params=pltpu.CompilerParams(
            needs_layout_passes=False, use_tc_tiling_on_sc=False),
        out_type=jax.ShapeDtypeStruct((NW, WORDS_PER_W), jnp.float32),
        scratch_types=[
            pltpu.VMEM((N, 4), jnp.float32),
            pltpu.VMEM((PAIRS_PER_W,), jnp.int32),
            pltpu.VMEM((WORDS_PER_W,), jnp.float32),
        ],
    )(_sc_gather_body)
    out2 = sc_gather(tab, idx2)

    neighborhood = out2.reshape(B, G, K, 4)[..., :3]
    return (neighborhood, pcd)


# speedup vs baseline: 1.0677x; 1.0218x over previous
import functools

import jax
import jax.numpy as jnp
from jax import lax
from jax.experimental import pallas as pl
from jax.experimental.pallas import tpu as pltpu
from jax.experimental.pallas import tpu_sc as plsc

N = 2048
G = 2048
K = 16
G_TILE = 512
NW = 32
PAIRS_PER_W = (G * K) // 2
WORDS_PER_W = PAIRS_PER_W * 3


def _knn_idx_body(pcd_ref, pfull_ref, xt_ref, out_ref):
    ct = pcd_ref[0]
    pf = pfull_ref[0]
    xt = xt_ref[0]
    sq = jnp.sum(xt * xt, axis=0, keepdims=True)
    sqg = jnp.sum(ct * ct, axis=1, keepdims=True)
    dots = jax.lax.dot_general(ct, pf, (((1,), (1,)), ((), ())),
                               preferred_element_type=jnp.float32)
    d2 = (sqg + sq) - 2.0 * dots
    ii = lax.broadcasted_iota(jnp.int32, (G_TILE, N), 1)
    idxs = []
    for _ in range(K):
        m = jnp.min(d2, axis=1, keepdims=True)
        cand = jnp.where(d2 == m, ii, N)
        im = jnp.min(cand, axis=1, keepdims=True)
        onehot = ii == im
        idxs.append(im)
        d2 = jnp.where(onehot, jnp.inf, d2)
    out_ref[0] = jnp.concatenate(idxs, axis=1)


def _sc_gather_body(tab_hbm, idx_hbm, out_hbm, tab_v, idx_v, out_v):
    wid = lax.axis_index("s") * 2 + lax.axis_index("c")
    b = wid >> 1
    half = wid & 1
    pltpu.sync_copy(tab_hbm.at[b], tab_v)
    pltpu.sync_copy(idx_hbm.at[wid], idx_v)
    lanes = lax.broadcasted_iota(jnp.int32, (16,), 0)
    gbase = half * PAIRS_PER_W

    def body(i, carry):
        g = (gbase >> 4) + i
        gvec = jnp.broadcast_to(g, (16,))
        for v in range(3):
            w = jnp.broadcast_to(i * 48 + v * 16, (16,)) + lanes
            pair = w // 3
            comp = w - pair * 3
            rows = plsc.load_gather(idx_v, [pair])
            val = plsc.load_gather(tab_v, [rows, comp])
            ctr = plsc.load_gather(tab_v, [gvec, comp])
            out_v[pl.ds(i * 48 + v * 16, 16)] = val - ctr
        return carry

    lax.fori_loop(0, PAIRS_PER_W // K, body, 0)
    pltpu.sync_copy(out_v, out_hbm.at[wid])


@jax.jit
def kernel(pcd):
    B = pcd.shape[0]
    xt = jnp.transpose(pcd, (0, 2, 1))
    idx = pl.pallas_call(
        _knn_idx_body,
        grid=(B, G // G_TILE),
        in_specs=[
            pl.BlockSpec((1, G_TILE, 3), lambda b, j: (b, j, 0)),
            pl.BlockSpec((1, N, 3), lambda b, j: (b, 0, 0)),
            pl.BlockSpec((1, 3, N), lambda b, j: (b, 0, 0)),
        ],
        out_specs=pl.BlockSpec((1, G_TILE, K), lambda b, j: (b, j, 0)),
        out_shape=jax.ShapeDtypeStruct((B, G, K), jnp.int32),
    )(pcd, pcd, xt)

    idx2 = idx.reshape(NW, PAIRS_PER_W)

    mesh = plsc.VectorSubcoreMesh(core_axis_name="c", subcore_axis_name="s")
    sc_gather = functools.partial(
        pl.kernel, mesh=mesh,
        compiler_params=pltpu.CompilerParams(
            needs_layout_passes=False, use_tc_tiling_on_sc=False),
        out_type=jax.ShapeDtypeStruct((NW, WORDS_PER_W), jnp.float32),
        scratch_types=[
            pltpu.VMEM((N, 3), jnp.float32),
            pltpu.VMEM((PAIRS_PER_W,), jnp.int32),
            pltpu.VMEM((WORDS_PER_W,), jnp.float32),
        ],
    )(_sc_gather_body)
    out2 = sc_gather(pcd, idx2)

    neighborhood = out2.reshape(B, G, K, 3)
    return (neighborhood, pcd)
